# baseline (speedup 1.0000x reference)
import functools

import jax
import jax.numpy as jnp
from jax import lax
from jax.experimental import pallas as pl
from jax.experimental.pallas import tpu as pltpu
from jax.experimental.pallas import tpu_sc as plsc

N = 10000
E = 320000
DF = 128
DE = 16
H = 256
EO = 128
NO = 128

NC = 2
NS = 16
NW = NC * NS

C = 128
K2CH = 80
PER_W = C * K2CH
EP = NW * PER_W
NP = 10240
TE = 2048
TN = 1024

_f32 = jnp.float32
_bf16 = jnp.bfloat16


def _pack2(a):
    lo = lax.bitcast_convert_type(a[:, :128].astype(_bf16), jnp.uint16)
    hi = lax.bitcast_convert_type(a[:, 128:].astype(_bf16), jnp.uint16)
    packed = hi.astype(jnp.uint32) << 16 | lo.astype(jnp.uint32)
    return lax.bitcast_convert_type(packed, jnp.int32)


def _proj_body(x_ref, wa_ref, wb_ref, wc_ref, oa_ref, ob_ref, oc_ref):
    xb = x_ref[...]
    a = jnp.dot(xb, wa_ref[...], preferred_element_type=_f32)
    b = jnp.dot(xb, wb_ref[...], preferred_element_type=_f32)
    oa_ref[...] = _pack2(a)
    ob_ref[...] = _pack2(b)
    oc_ref[...] = jnp.dot(xb, wc_ref[...], preferred_element_type=_f32)


def _proj(x_p, w1a, w1b, wn1a):
    grid = NP // TN
    return pl.pallas_call(
        _proj_body,
        grid=(grid,),
        in_specs=[
            pl.BlockSpec((TN, DF), lambda i: (i, 0)),
            pl.BlockSpec((DF, H), lambda i: (0, 0)),
            pl.BlockSpec((DF, H), lambda i: (0, 0)),
            pl.BlockSpec((DF, H), lambda i: (0, 0)),
        ],
        out_specs=[
            pl.BlockSpec((TN, 128), lambda i: (i, 0)),
            pl.BlockSpec((TN, 128), lambda i: (i, 0)),
            pl.BlockSpec((TN, H), lambda i: (i, 0)),
        ],
        out_shape=[
            jax.ShapeDtypeStruct((NP, 128), jnp.int32),
            jax.ShapeDtypeStruct((NP, 128), jnp.int32),
            jax.ShapeDtypeStruct((NP, H), _f32),
        ],
    )(x_p, w1a, w1b, wn1a)


CH_FAST = 128
CH_SLOW = 32
MAXCH2 = CH_FAST // 2


def _gather_body(xs_hbm, xr_hbm, src2_hbm, dst2_hbm, gs_hbm, gr_hbm,
                 idx_s, idx_d, buf_a0, buf_b0, buf_a1, buf_b1,
                 sem_a0, sem_b0, sem_a1, sem_b1, sem_o0, sem_o1):
    cid = lax.axis_index("c")
    sid = lax.axis_index("s")
    my_ch = jnp.where(cid == 0, CH_FAST, CH_SLOW)
    ch0 = jnp.where(cid == 0, sid * CH_FAST, NS * CH_FAST + sid * CH_SLOW)
    nb2 = my_ch // 2

    pltpu.sync_copy(src2_hbm.at[pl.ds(pl.multiple_of(ch0, 8), MAXCH2 * 2)], idx_s)
    pltpu.sync_copy(dst2_hbm.at[pl.ds(pl.multiple_of(ch0, 8), MAXCH2 * 2)], idx_d)

    def row_off(j):
        return pl.multiple_of((ch0 + j) * C, 8)

    def issue(j, buf_a, buf_b, sem_a, sem_b):
        pltpu.async_copy(xs_hbm.at[idx_s.at[j]], buf_a, sem_a)
        pltpu.async_copy(xr_hbm.at[idx_d.at[j]], buf_b, sem_b)

    def wait_gather(j, buf_a, buf_b, sem_a, sem_b):
        pltpu.make_async_copy(xs_hbm.at[idx_s.at[j]], buf_a, sem_a).wait()
        pltpu.make_async_copy(xr_hbm.at[idx_d.at[j]], buf_b, sem_b).wait()

    def issue_out(j, buf_a, buf_b, sem_o):
        pltpu.async_copy(buf_a, gs_hbm.at[pl.ds(row_off(j), C)], sem_o)
        pltpu.async_copy(buf_b, gr_hbm.at[pl.ds(row_off(j), C)], sem_o)

    def wait_out(j, buf_a, buf_b, sem_o):
        pltpu.make_async_copy(buf_a, gs_hbm.at[pl.ds(row_off(j), C)], sem_o).wait()
        pltpu.make_async_copy(buf_b, gr_hbm.at[pl.ds(row_off(j), C)], sem_o).wait()

    issue(0, buf_a0, buf_b0, sem_a0, sem_b0)
    issue(1, buf_a1, buf_b1, sem_a1, sem_b1)

    def pair(jj, carry):
        j0 = jj * 2
        j1 = j0 + 1
        wait_gather(j0, buf_a0, buf_b0, sem_a0, sem_b0)
        issue_out(j0, buf_a0, buf_b0, sem_o0)
        wait_gather(j1, buf_a1, buf_b1, sem_a1, sem_b1)
        issue_out(j1, buf_a1, buf_b1, sem_o1)
        wait_out(j0, buf_a0, buf_b0, sem_o0)

        @pl.when(jj + 1 < nb2)
        def _():
            issue(j0 + 2, buf_a0, buf_b0, sem_a0, sem_b0)

        wait_out(j1, buf_a1, buf_b1, sem_o1)

        @pl.when(jj + 1 < nb2)
        def _():
            issue(j1 + 2, buf_a1, buf_b1, sem_a1, sem_b1)

        return carry

    lax.fori_loop(0, nb2, pair, 0)


def _gather(xs, xr, src2, dst2):
    mesh = plsc.VectorSubcoreMesh(core_axis_name="c", subcore_axis_name="s")
    fn = functools.partial(
        pl.kernel,
        out_type=[jax.ShapeDtypeStruct((EP, 128), jnp.int32),
                  jax.ShapeDtypeStruct((EP, 128), jnp.int32)],
        mesh=mesh,
        scratch_types=[
            pltpu.VMEM((MAXCH2 * 2, C), jnp.int32),
            pltpu.VMEM((MAXCH2 * 2, C), jnp.int32),
            pltpu.VMEM((C, 128), jnp.int32),
            pltpu.VMEM((C, 128), jnp.int32),
            pltpu.VMEM((C, 128), jnp.int32),
            pltpu.VMEM((C, 128), jnp.int32),
            pltpu.SemaphoreType.DMA,
            pltpu.SemaphoreType.DMA,
            pltpu.SemaphoreType.DMA,
            pltpu.SemaphoreType.DMA,
            pltpu.SemaphoreType.DMA,
            pltpu.SemaphoreType.DMA,
        ],
    )(_gather_body)
    return fn(xs, xr, src2, dst2)


def _unpack2(gi):
    lo16 = (gi & 0xFFFF).astype(jnp.uint16)
    hi16 = ((gi >> 16) & 0xFFFF).astype(jnp.uint16)
    return (lax.bitcast_convert_type(lo16, _bf16).astype(_f32),
            lax.bitcast_convert_type(hi16, _bf16).astype(_f32))


def _edge_body(gs_ref, gr_ref, ea_ref, w1c_ref, b1_ref, w2a_ref, w2b_ref,
               b2_ref, w3_ref, b3_ref, out_ref):
    ea1 = jnp.dot(ea_ref[...], w1c_ref[...],
                  preferred_element_type=_f32) + b1_ref[...]
    sa, sb = _unpack2(gs_ref[...])
    ra, rb = _unpack2(gr_ref[...])
    h1a = jnp.maximum(sa + ra + ea1[:, :128], 0.0)
    h1b = jnp.maximum(sb + rb + ea1[:, 128:], 0.0)
    h2 = jnp.dot(h1a, w2a_ref[...], preferred_element_type=_f32) + \
         jnp.dot(h1b, w2b_ref[...], preferred_element_type=_f32) + b2_ref[...]
    h2 = jnp.maximum(h2, 0.0)
    out_ref[...] = jnp.dot(h2, w3_ref[...],
                           preferred_element_type=_f32) + b3_ref[...]


def _edge_mlp(gs, gr, ea_p, w1c, b1, w2a, w2b, b2, w3, b3):
    grid = EP // TE
    return pl.pallas_call(
        _edge_body,
        grid=(grid,),
        in_specs=[
            pl.BlockSpec((TE, 128), lambda i: (i, 0)),
            pl.BlockSpec((TE, 128), lambda i: (i, 0)),
            pl.BlockSpec((TE, DE), lambda i: (i, 0)),
            pl.BlockSpec((DE, H), lambda i: (0, 0)),
            pl.BlockSpec((1, H), lambda i: (0, 0)),
            pl.BlockSpec((128, H), lambda i: (0, 0)),
            pl.BlockSpec((128, H), lambda i: (0, 0)),
            pl.BlockSpec((1, H), lambda i: (0, 0)),
            pl.BlockSpec((H, EO), lambda i: (0, 0)),
            pl.BlockSpec((1, EO), lambda i: (0, 0)),
        ],
        out_specs=pl.BlockSpec((TE, EO), lambda i: (i, 0)),
        out_shape=jax.ShapeDtypeStruct((EP, EO), _f32),
    )(gs, gr, ea_p, w1c, b1, w2a, w2b, b2, w3, b3)


ROWS_PER_TILE = NP // NS
CZ = 128


def _scatter_body(eo_hbm, dst2_hbm, out_hbm, idxs, buf, zbuf, accum, sem):
    cid = lax.axis_index("c")
    sid = lax.axis_index("s")
    wid = sid * NC + cid

    def zrow(r, carry):
        for t in range(EO // 16):
            zbuf[r, pl.ds(t * 16, 16)] = jnp.zeros((16,), _f32)
        return carry

    lax.fori_loop(0, CZ, zrow, 0)
    for z in range(ROWS_PER_TILE // CZ):
        r0 = pl.multiple_of(sid * ROWS_PER_TILE + z * CZ, 8)
        pltpu.sync_copy(zbuf, accum.at[pl.ds(r0, CZ)])
    plsc.subcore_barrier()

    pltpu.sync_copy(dst2_hbm.at[pl.ds(wid * K2CH, K2CH)], idxs)

    def chunk(j, carry):
        eb = pl.multiple_of(wid * PER_W + j * C, 8)
        pltpu.sync_copy(eo_hbm.at[pl.ds(eb, C)], buf)
        pltpu.sync_copy(buf, accum.at[idxs.at[j]], add=True)
        return carry

    lax.fori_loop(0, K2CH, chunk, 0)
    plsc.subcore_barrier()

    for z in range(ROWS_PER_TILE // CZ):
        r0 = pl.multiple_of(sid * ROWS_PER_TILE + z * CZ, 8)
        pltpu.sync_copy(accum.at[pl.ds(r0, CZ)], out_hbm.at[cid].at[pl.ds(r0, CZ)])


def _scatter(e_out_p, dst2):
    mesh = plsc.VectorSubcoreMesh(core_axis_name="c", subcore_axis_name="s")
    fn = functools.partial(
        pl.kernel,
        out_type=jax.ShapeDtypeStruct((NC, NP, EO), _f32),
        mesh=mesh,
        scratch_types=[
            pltpu.VMEM((K2CH, C), jnp.int32),
            pltpu.VMEM((C, EO), _f32),
            pltpu.VMEM((CZ, EO), _f32),
            pltpu.VMEM_SHARED((NP, EO), _f32),
            pltpu.SemaphoreType.DMA,
        ],
    )(_scatter_body)
    return fn(e_out_p, dst2)


def _node_body(a0_ref, a1_ref, xn1_ref, w1b_ref, b1_ref, w2_ref, b2_ref,
               w3_ref, b3_ref, out_ref):
    ie = a0_ref[...] + a1_ref[...]
    h1 = xn1_ref[...] + jnp.dot(ie, w1b_ref[...],
                                preferred_element_type=_f32) + b1_ref[...]
    h1 = jnp.maximum(h1, 0.0)
    h2 = jnp.dot(h1, w2_ref[...], preferred_element_type=_f32) + b2_ref[...]
    h2 = jnp.maximum(h2, 0.0)
    out_ref[...] = jnp.dot(h2, w3_ref[...],
                           preferred_element_type=_f32) + b3_ref[...]


def _node_mlp(acc0, acc1, xn1, wn1b, b1, w2, b2, w3, b3):
    grid = NP // TN
    return pl.pallas_call(
        _node_body,
        grid=(grid,),
        in_specs=[
            pl.BlockSpec((TN, EO), lambda i: (i, 0)),
            pl.BlockSpec((TN, EO), lambda i: (i, 0)),
            pl.BlockSpec((TN, H), lambda i: (i, 0)),
            pl.BlockSpec((EO, H), lambda i: (0, 0)),
            pl.BlockSpec((1, H), lambda i: (0, 0)),
            pl.BlockSpec((H, H), lambda i: (0, 0)),
            pl.BlockSpec((1, H), lambda i: (0, 0)),
            pl.BlockSpec((H, NO), lambda i: (0, 0)),
            pl.BlockSpec((1, NO), lambda i: (0, 0)),
        ],
        out_specs=pl.BlockSpec((TN, NO), lambda i: (i, 0)),
        out_shape=jax.ShapeDtypeStruct((NP, NO), _f32),
    )(acc0, acc1, xn1, wn1b, b1, w2, b2, w3, b3)


def kernel(x, edge_index, edge_attr, fe_W1, fe_b1, fe_W2, fe_b2, fe_W3, fe_b3,
           fn_W1, fn_b1, fn_W2, fn_b2, fn_W3, fn_b3):
    x_p = jnp.pad(x, ((0, NP - N), (0, 0)))
    src_p = jnp.pad(edge_index[0], (0, EP - E), constant_values=N)
    dst_p = jnp.pad(edge_index[1], (0, EP - E), constant_values=N)
    ea_p = jnp.pad(edge_attr, ((0, EP - E), (0, 0)))

    w1a = fe_W1[:DF]
    w1b = fe_W1[DF:2 * DF]
    w1c = fe_W1[2 * DF:]
    wn1a = fn_W1[:DF]
    wn1b = fn_W1[DF:]

    xs, xr, xn1 = _proj(x_p, w1a, w1b, wn1a)
    pad_rows = ((0, MAXCH2 * 2 - CH_SLOW), (0, 0))
    src2 = jnp.pad(src_p.reshape(EP // C, C), pad_rows, constant_values=N)
    dst2 = jnp.pad(dst_p.reshape(EP // C, C), pad_rows, constant_values=N)
    gs, gr = _gather(xs, xr, src2, dst2)
    e_out_p = _edge_mlp(gs, gr, ea_p, w1c, fe_b1[None, :], fe_W2[:128],
                        fe_W2[128:], fe_b2[None, :], fe_W3, fe_b3[None, :])
    acc = _scatter(e_out_p, dst_p.reshape(EP // C, C))
    n_out_p = _node_mlp(acc[0], acc[1], xn1, wn1b, fn_b1[None, :], fn_W2,
                        fn_b2[None, :], fn_W3, fn_b3[None, :])
    return n_out_p[:N], e_out_p[:E]


# speedup vs baseline: 1.4337x; 1.4337x over previous
import functools

import jax
import jax.numpy as jnp
from jax import lax
from jax.experimental import pallas as pl
from jax.experimental.pallas import tpu as pltpu
from jax.experimental.pallas import tpu_sc as plsc

N = 10000
E = 320000
DF = 128
DE = 16
H = 256
EO = 128
NO = 128

NC = 2
NS = 16
NW = NC * NS

C = 128
K2CH = 80
PER_W = C * K2CH
EP = NW * PER_W
NP = 10240
TE = 2048
TN = 1024

_f32 = jnp.float32
_bf16 = jnp.bfloat16


def _pack2(a):
    lo = lax.bitcast_convert_type(a[:, :128].astype(_bf16), jnp.uint16)
    hi = lax.bitcast_convert_type(a[:, 128:].astype(_bf16), jnp.uint16)
    packed = hi.astype(jnp.uint32) << 16 | lo.astype(jnp.uint32)
    return lax.bitcast_convert_type(packed, jnp.int32)


def _proj_body(x_ref, wa_ref, wb_ref, wc_ref, oa_ref, ob_ref, oc_ref):
    xb = x_ref[...]
    a = jnp.dot(xb, wa_ref[...], preferred_element_type=_f32)
    b = jnp.dot(xb, wb_ref[...], preferred_element_type=_f32)
    oa_ref[...] = _pack2(a)
    ob_ref[...] = _pack2(b)
    oc_ref[...] = jnp.dot(xb, wc_ref[...], preferred_element_type=_f32)


def _proj(x_p, w1a, w1b, wn1a):
    grid = NP // TN
    return pl.pallas_call(
        _proj_body,
        grid=(grid,),
        in_specs=[
            pl.BlockSpec((TN, DF), lambda i: (i, 0)),
            pl.BlockSpec((DF, H), lambda i: (0, 0)),
            pl.BlockSpec((DF, H), lambda i: (0, 0)),
            pl.BlockSpec((DF, H), lambda i: (0, 0)),
        ],
        out_specs=[
            pl.BlockSpec((TN, 128), lambda i: (i, 0)),
            pl.BlockSpec((TN, 128), lambda i: (i, 0)),
            pl.BlockSpec((TN, H), lambda i: (i, 0)),
        ],
        out_shape=[
            jax.ShapeDtypeStruct((NP, 128), jnp.int32),
            jax.ShapeDtypeStruct((NP, 128), jnp.int32),
            jax.ShapeDtypeStruct((NP, H), _f32),
        ],
    )(x_p, w1a, w1b, wn1a)


CH_FAST = 80
CH_SLOW = 80
MAXCH2 = CH_FAST // 2


def _gather_body(xs_hbm, xr_hbm, src2_hbm, dst2_hbm, gs_hbm, gr_hbm,
                 idx_s, idx_d, buf_a0, buf_b0, buf_a1, buf_b1,
                 sem_a0, sem_b0, sem_a1, sem_b1, sem_o0, sem_o1):
    cid = lax.axis_index("c")
    sid = lax.axis_index("s")
    my_ch = jnp.where(cid == 0, CH_FAST, CH_SLOW)
    ch0 = jnp.where(cid == 0, sid * CH_FAST, NS * CH_FAST + sid * CH_SLOW)
    nb2 = my_ch // 2

    pltpu.sync_copy(src2_hbm.at[pl.ds(pl.multiple_of(ch0, 8), MAXCH2 * 2)], idx_s)
    pltpu.sync_copy(dst2_hbm.at[pl.ds(pl.multiple_of(ch0, 8), MAXCH2 * 2)], idx_d)

    def row_off(j):
        return pl.multiple_of((ch0 + j) * C, 8)

    def issue(j, buf_a, buf_b, sem_a, sem_b):
        pltpu.async_copy(xs_hbm.at[idx_s.at[j]], buf_a, sem_a)
        pltpu.async_copy(xr_hbm.at[idx_d.at[j]], buf_b, sem_b)

    def wait_gather(j, buf_a, buf_b, sem_a, sem_b):
        pltpu.make_async_copy(xs_hbm.at[idx_s.at[j]], buf_a, sem_a).wait()
        pltpu.make_async_copy(xr_hbm.at[idx_d.at[j]], buf_b, sem_b).wait()

    def issue_out(j, buf_a, buf_b, sem_o):
        pltpu.async_copy(buf_a, gs_hbm.at[pl.ds(row_off(j), C)], sem_o)
        pltpu.async_copy(buf_b, gr_hbm.at[pl.ds(row_off(j), C)], sem_o)

    def wait_out(j, buf_a, buf_b, sem_o):
        pltpu.make_async_copy(buf_a, gs_hbm.at[pl.ds(row_off(j), C)], sem_o).wait()
        pltpu.make_async_copy(buf_b, gr_hbm.at[pl.ds(row_off(j), C)], sem_o).wait()

    issue(0, buf_a0, buf_b0, sem_a0, sem_b0)
    issue(1, buf_a1, buf_b1, sem_a1, sem_b1)

    def pair(jj, carry):
        j0 = jj * 2
        j1 = j0 + 1
        wait_gather(j0, buf_a0, buf_b0, sem_a0, sem_b0)
        issue_out(j0, buf_a0, buf_b0, sem_o0)
        wait_gather(j1, buf_a1, buf_b1, sem_a1, sem_b1)
        issue_out(j1, buf_a1, buf_b1, sem_o1)
        wait_out(j0, buf_a0, buf_b0, sem_o0)

        @pl.when(jj + 1 < nb2)
        def _():
            issue(j0 + 2, buf_a0, buf_b0, sem_a0, sem_b0)

        wait_out(j1, buf_a1, buf_b1, sem_o1)

        @pl.when(jj + 1 < nb2)
        def _():
            issue(j1 + 2, buf_a1, buf_b1, sem_a1, sem_b1)

        return carry

    lax.fori_loop(0, nb2, pair, 0)


def _gather(xs, xr, src2, dst2):
    mesh = plsc.VectorSubcoreMesh(core_axis_name="c", subcore_axis_name="s")
    fn = functools.partial(
        pl.kernel,
        out_type=[jax.ShapeDtypeStruct((EP, 128), jnp.int32),
                  jax.ShapeDtypeStruct((EP, 128), jnp.int32)],
        mesh=mesh,
        scratch_types=[
            pltpu.VMEM((MAXCH2 * 2, C), jnp.int32),
            pltpu.VMEM((MAXCH2 * 2, C), jnp.int32),
            pltpu.VMEM((C, 128), jnp.int32),
            pltpu.VMEM((C, 128), jnp.int32),
            pltpu.VMEM((C, 128), jnp.int32),
            pltpu.VMEM((C, 128), jnp.int32),
            pltpu.SemaphoreType.DMA,
            pltpu.SemaphoreType.DMA,
            pltpu.SemaphoreType.DMA,
            pltpu.SemaphoreType.DMA,
            pltpu.SemaphoreType.DMA,
            pltpu.SemaphoreType.DMA,
        ],
    )(_gather_body)
    return fn(xs, xr, src2, dst2)


def _unpack2(gi):
    lo16 = (gi & 0xFFFF).astype(jnp.uint16)
    hi16 = ((gi >> 16) & 0xFFFF).astype(jnp.uint16)
    return (lax.bitcast_convert_type(lo16, _bf16).astype(_f32),
            lax.bitcast_convert_type(hi16, _bf16).astype(_f32))


def _edge_body(gs_ref, gr_ref, ea_ref, w1c_ref, b1_ref, w2a_ref, w2b_ref,
               b2_ref, w3_ref, b3_ref, out_ref):
    ea1 = jnp.dot(ea_ref[...], w1c_ref[...],
                  preferred_element_type=_f32) + b1_ref[...]
    sa, sb = _unpack2(gs_ref[...])
    ra, rb = _unpack2(gr_ref[...])
    h1a = jnp.maximum(sa + ra + ea1[:, :128], 0.0)
    h1b = jnp.maximum(sb + rb + ea1[:, 128:], 0.0)
    h2 = jnp.dot(h1a, w2a_ref[...], preferred_element_type=_f32) + \
         jnp.dot(h1b, w2b_ref[...], preferred_element_type=_f32) + b2_ref[...]
    h2 = jnp.maximum(h2, 0.0)
    out_ref[...] = jnp.dot(h2, w3_ref[...],
                           preferred_element_type=_f32) + b3_ref[...]


def _edge_mlp(gs, gr, ea_p, w1c, b1, w2a, w2b, b2, w3, b3):
    grid = EP // TE
    return pl.pallas_call(
        _edge_body,
        grid=(grid,),
        in_specs=[
            pl.BlockSpec((TE, 128), lambda i: (i, 0)),
            pl.BlockSpec((TE, 128), lambda i: (i, 0)),
            pl.BlockSpec((TE, DE), lambda i: (i, 0)),
            pl.BlockSpec((DE, H), lambda i: (0, 0)),
            pl.BlockSpec((1, H), lambda i: (0, 0)),
            pl.BlockSpec((128, H), lambda i: (0, 0)),
            pl.BlockSpec((128, H), lambda i: (0, 0)),
            pl.BlockSpec((1, H), lambda i: (0, 0)),
            pl.BlockSpec((H, EO), lambda i: (0, 0)),
            pl.BlockSpec((1, EO), lambda i: (0, 0)),
        ],
        out_specs=pl.BlockSpec((TE, EO), lambda i: (i, 0)),
        out_shape=jax.ShapeDtypeStruct((EP, EO), _f32),
    )(gs, gr, ea_p, w1c, b1, w2a, w2b, b2, w3, b3)


ROWS_PER_TILE = NP // NS
CZ = 128


def _scatter_body(eo_hbm, dst2_hbm, out_hbm, idxs, buf, zbuf, accum, sem):
    cid = lax.axis_index("c")
    sid = lax.axis_index("s")
    wid = sid * NC + cid

    def zrow(r, carry):
        for t in range(EO // 16):
            zbuf[r, pl.ds(t * 16, 16)] = jnp.zeros((16,), _f32)
        return carry

    lax.fori_loop(0, CZ, zrow, 0)
    for z in range(ROWS_PER_TILE // CZ):
        r0 = pl.multiple_of(sid * ROWS_PER_TILE + z * CZ, 8)
        pltpu.sync_copy(zbuf, accum.at[pl.ds(r0, CZ)])
    plsc.subcore_barrier()

    pltpu.sync_copy(dst2_hbm.at[pl.ds(wid * K2CH, K2CH)], idxs)

    def chunk(j, carry):
        eb = pl.multiple_of(wid * PER_W + j * C, 8)
        pltpu.sync_copy(eo_hbm.at[pl.ds(eb, C)], buf)
        pltpu.sync_copy(buf, accum.at[idxs.at[j]], add=True)
        return carry

    lax.fori_loop(0, K2CH, chunk, 0)
    plsc.subcore_barrier()

    for z in range(ROWS_PER_TILE // CZ):
        r0 = pl.multiple_of(sid * ROWS_PER_TILE + z * CZ, 8)
        pltpu.sync_copy(accum.at[pl.ds(r0, CZ)], out_hbm.at[cid].at[pl.ds(r0, CZ)])


def _scatter(e_out_p, dst2):
    mesh = plsc.VectorSubcoreMesh(core_axis_name="c", subcore_axis_name="s")
    fn = functools.partial(
        pl.kernel,
        out_type=jax.ShapeDtypeStruct((NC, NP, EO), _f32),
        mesh=mesh,
        scratch_types=[
            pltpu.VMEM((K2CH, C), jnp.int32),
            pltpu.VMEM((C, EO), _f32),
            pltpu.VMEM((CZ, EO), _f32),
            pltpu.VMEM_SHARED((NP, EO), _f32),
            pltpu.SemaphoreType.DMA,
        ],
    )(_scatter_body)
    return fn(e_out_p, dst2)


def _node_body(a0_ref, a1_ref, xn1_ref, w1b_ref, b1_ref, w2_ref, b2_ref,
               w3_ref, b3_ref, out_ref):
    ie = a0_ref[...] + a1_ref[...]
    h1 = xn1_ref[...] + jnp.dot(ie, w1b_ref[...],
                                preferred_element_type=_f32) + b1_ref[...]
    h1 = jnp.maximum(h1, 0.0)
    h2 = jnp.dot(h1, w2_ref[...], preferred_element_type=_f32) + b2_ref[...]
    h2 = jnp.maximum(h2, 0.0)
    out_ref[...] = jnp.dot(h2, w3_ref[...],
                           preferred_element_type=_f32) + b3_ref[...]


def _node_mlp(acc0, acc1, xn1, wn1b, b1, w2, b2, w3, b3):
    grid = NP // TN
    return pl.pallas_call(
        _node_body,
        grid=(grid,),
        in_specs=[
            pl.BlockSpec((TN, EO), lambda i: (i, 0)),
            pl.BlockSpec((TN, EO), lambda i: (i, 0)),
            pl.BlockSpec((TN, H), lambda i: (i, 0)),
            pl.BlockSpec((EO, H), lambda i: (0, 0)),
            pl.BlockSpec((1, H), lambda i: (0, 0)),
            pl.BlockSpec((H, H), lambda i: (0, 0)),
            pl.BlockSpec((1, H), lambda i: (0, 0)),
            pl.BlockSpec((H, NO), lambda i: (0, 0)),
            pl.BlockSpec((1, NO), lambda i: (0, 0)),
        ],
        out_specs=pl.BlockSpec((TN, NO), lambda i: (i, 0)),
        out_shape=jax.ShapeDtypeStruct((NP, NO), _f32),
    )(acc0, acc1, xn1, wn1b, b1, w2, b2, w3, b3)


def kernel(x, edge_index, edge_attr, fe_W1, fe_b1, fe_W2, fe_b2, fe_W3, fe_b3,
           fn_W1, fn_b1, fn_W2, fn_b2, fn_W3, fn_b3):
    x_p = jnp.pad(x, ((0, NP - N), (0, 0)))
    src_p = jnp.pad(edge_index[0], (0, EP - E), constant_values=N)
    dst_p = jnp.pad(edge_index[1], (0, EP - E), constant_values=N)
    ea_p = jnp.pad(edge_attr, ((0, EP - E), (0, 0)))

    w1a = fe_W1[:DF]
    w1b = fe_W1[DF:2 * DF]
    w1c = fe_W1[2 * DF:]
    wn1a = fn_W1[:DF]
    wn1b = fn_W1[DF:]

    xs, xr, xn1 = _proj(x_p, w1a, w1b, wn1a)
    spread = jnp.arange(EP - E, dtype=jnp.int32) % N
    src_g = jnp.concatenate([edge_index[0], spread])
    dst_g = jnp.concatenate([edge_index[1], spread])
    pad_rows = ((0, MAXCH2 * 2 - CH_SLOW), (0, 0))
    src2 = jnp.pad(src_g.reshape(EP // C, C), pad_rows)
    dst2 = jnp.pad(dst_g.reshape(EP // C, C), pad_rows)
    gs, gr = _gather(xs, xr, src2, dst2)
    e_out_p = _edge_mlp(gs, gr, ea_p, w1c, fe_b1[None, :], fe_W2[:128],
                        fe_W2[128:], fe_b2[None, :], fe_W3, fe_b3[None, :])
    acc = _scatter(e_out_p, dst_p.reshape(EP // C, C))
    n_out_p = _node_mlp(acc[0], acc[1], xn1, wn1b, fn_b1[None, :], fn_W2,
                        fn_b2[None, :], fn_W3, fn_b3[None, :])
    return n_out_p[:N], e_out_p[:E]


# speedup vs baseline: 1.5729x; 1.0971x over previous
import functools

import jax
import jax.numpy as jnp
from jax import lax
from jax.experimental import pallas as pl
from jax.experimental.pallas import tpu as pltpu
from jax.experimental.pallas import tpu_sc as plsc

N = 10000
E = 320000
DF = 128
DE = 16
H = 256
EO = 128
NO = 128

NC = 2
NS = 16
NW = NC * NS

C = 128
K2CH = 80
PER_W = C * K2CH
EP = NW * PER_W
NP = 10240
TE = 2048
TN = 1024

NPH = 2
EPH = EP // NPH
PH_CH = K2CH // NPH
PH_ROWS = EPH // C
PH_TILES = EPH // TE

_f32 = jnp.float32
_bf16 = jnp.bfloat16


def _pack2(a):
    lo = lax.bitcast_convert_type(a[:, :128].astype(_bf16), jnp.uint16)
    hi = lax.bitcast_convert_type(a[:, 128:].astype(_bf16), jnp.uint16)
    packed = hi.astype(jnp.uint32) << 16 | lo.astype(jnp.uint32)
    return lax.bitcast_convert_type(packed, jnp.int32)


def _proj_body(x_ref, wa_ref, wb_ref, wc_ref, oa_ref, ob_ref, oc_ref):
    xb = x_ref[...]
    a = jnp.dot(xb, wa_ref[...], preferred_element_type=_f32)
    b = jnp.dot(xb, wb_ref[...], preferred_element_type=_f32)
    oa_ref[...] = _pack2(a)
    ob_ref[...] = _pack2(b)
    oc_ref[...] = jnp.dot(xb, wc_ref[...], preferred_element_type=_f32)


def _proj(x_p, w1a, w1b, wn1a):
    grid = NP // TN
    return pl.pallas_call(
        _proj_body,
        grid=(grid,),
        in_specs=[
            pl.BlockSpec((TN, DF), lambda i: (i, 0)),
            pl.BlockSpec((DF, H), lambda i: (0, 0)),
            pl.BlockSpec((DF, H), lambda i: (0, 0)),
            pl.BlockSpec((DF, H), lambda i: (0, 0)),
        ],
        out_specs=[
            pl.BlockSpec((TN, 128), lambda i: (i, 0)),
            pl.BlockSpec((TN, 128), lambda i: (i, 0)),
            pl.BlockSpec((TN, H), lambda i: (i, 0)),
        ],
        out_shape=[
            jax.ShapeDtypeStruct((NP, 128), jnp.int32),
            jax.ShapeDtypeStruct((NP, 128), jnp.int32),
            jax.ShapeDtypeStruct((NP, H), _f32),
        ],
    )(x_p, w1a, w1b, wn1a)


def _gather_body(xs_hbm, xr_hbm, src2_hbm, dst2_hbm, gs_hbm, gr_hbm,
                 idx_s, idx_d, buf_a0, buf_b0, buf_a1, buf_b1,
                 sem_a0, sem_b0, sem_a1, sem_b1, sem_o0, sem_o1):
    cid = lax.axis_index("c")
    sid = lax.axis_index("s")
    wid = sid * NC + cid
    ch0 = pl.multiple_of(wid * PH_CH, 8)

    pltpu.sync_copy(src2_hbm.at[pl.ds(ch0, PH_CH)], idx_s)
    pltpu.sync_copy(dst2_hbm.at[pl.ds(ch0, PH_CH)], idx_d)

    def row_off(j):
        return pl.multiple_of((ch0 + j) * C, 8)

    def issue(j, buf_a, buf_b, sem_a, sem_b):
        pltpu.async_copy(xs_hbm.at[idx_s.at[j]], buf_a, sem_a)
        pltpu.async_copy(xr_hbm.at[idx_d.at[j]], buf_b, sem_b)

    def wait_gather(j, buf_a, buf_b, sem_a, sem_b):
        pltpu.make_async_copy(xs_hbm.at[idx_s.at[j]], buf_a, sem_a).wait()
        pltpu.make_async_copy(xr_hbm.at[idx_d.at[j]], buf_b, sem_b).wait()

    def issue_out(j, buf_a, buf_b, sem_o):
        pltpu.async_copy(buf_a, gs_hbm.at[pl.ds(row_off(j), C)], sem_o)
        pltpu.async_copy(buf_b, gr_hbm.at[pl.ds(row_off(j), C)], sem_o)

    def wait_out(j, buf_a, buf_b, sem_o):
        pltpu.make_async_copy(buf_a, gs_hbm.at[pl.ds(row_off(j), C)], sem_o).wait()
        pltpu.make_async_copy(buf_b, gr_hbm.at[pl.ds(row_off(j), C)], sem_o).wait()

    issue(0, buf_a0, buf_b0, sem_a0, sem_b0)
    issue(1, buf_a1, buf_b1, sem_a1, sem_b1)
    nb2 = PH_CH // 2

    def pair(jj, carry):
        j0 = jj * 2
        j1 = j0 + 1
        wait_gather(j0, buf_a0, buf_b0, sem_a0, sem_b0)
        issue_out(j0, buf_a0, buf_b0, sem_o0)
        wait_gather(j1, buf_a1, buf_b1, sem_a1, sem_b1)
        issue_out(j1, buf_a1, buf_b1, sem_o1)
        wait_out(j0, buf_a0, buf_b0, sem_o0)

        @pl.when(jj + 1 < nb2)
        def _():
            issue(j0 + 2, buf_a0, buf_b0, sem_a0, sem_b0)

        wait_out(j1, buf_a1, buf_b1, sem_o1)

        @pl.when(jj + 1 < nb2)
        def _():
            issue(j1 + 2, buf_a1, buf_b1, sem_a1, sem_b1)

        return carry

    lax.fori_loop(0, nb2, pair, 0)


def _gather(xs, xr, src2, dst2):
    mesh = plsc.VectorSubcoreMesh(core_axis_name="c", subcore_axis_name="s")
    fn = functools.partial(
        pl.kernel,
        out_type=[jax.ShapeDtypeStruct((EPH, 128), jnp.int32),
                  jax.ShapeDtypeStruct((EPH, 128), jnp.int32)],
        mesh=mesh,
        scratch_types=[
            pltpu.VMEM((PH_CH, C), jnp.int32),
            pltpu.VMEM((PH_CH, C), jnp.int32),
            pltpu.VMEM((C, 128), jnp.int32),
            pltpu.VMEM((C, 128), jnp.int32),
            pltpu.VMEM((C, 128), jnp.int32),
            pltpu.VMEM((C, 128), jnp.int32),
            pltpu.SemaphoreType.DMA,
            pltpu.SemaphoreType.DMA,
            pltpu.SemaphoreType.DMA,
            pltpu.SemaphoreType.DMA,
            pltpu.SemaphoreType.DMA,
            pltpu.SemaphoreType.DMA,
        ],
    )(_gather_body)
    return fn(xs, xr, src2, dst2)


def _unpack2(gi):
    lo16 = (gi & 0xFFFF).astype(jnp.uint16)
    hi16 = ((gi >> 16) & 0xFFFF).astype(jnp.uint16)
    return (lax.bitcast_convert_type(lo16, _bf16).astype(_f32),
            lax.bitcast_convert_type(hi16, _bf16).astype(_f32))


def _edge_body(gs_ref, gr_ref, ea_ref, w1c_ref, b1_ref, w2a_ref, w2b_ref,
               b2_ref, w3_ref, b3_ref, *rest):
    out_ref = rest[-1]
    ea1 = jnp.dot(ea_ref[...], w1c_ref[...],
                  preferred_element_type=_f32) + b1_ref[...]
    sa, sb = _unpack2(gs_ref[...])
    ra, rb = _unpack2(gr_ref[...])
    h1a = jnp.maximum(sa + ra + ea1[:, :128], 0.0)
    h1b = jnp.maximum(sb + rb + ea1[:, 128:], 0.0)
    h2 = jnp.dot(h1a, w2a_ref[...], preferred_element_type=_f32) + \
         jnp.dot(h1b, w2b_ref[...], preferred_element_type=_f32) + b2_ref[...]
    h2 = jnp.maximum(h2, 0.0)
    out_ref[...] = jnp.dot(h2, w3_ref[...],
                           preferred_element_type=_f32) + b3_ref[...]


def _edge_mlp(gs, gr, ea_p, w1c, b1, w2a, w2b, b2, w3, b3, phase, prev=None):
    off = phase * PH_TILES
    in_specs = [
        pl.BlockSpec((TE, 128), lambda i: (i, 0)),
        pl.BlockSpec((TE, 128), lambda i: (i, 0)),
        pl.BlockSpec((TE, DE), lambda i: (i + off, 0)),
        pl.BlockSpec((DE, H), lambda i: (0, 0)),
        pl.BlockSpec((1, H), lambda i: (0, 0)),
        pl.BlockSpec((128, H), lambda i: (0, 0)),
        pl.BlockSpec((128, H), lambda i: (0, 0)),
        pl.BlockSpec((1, H), lambda i: (0, 0)),
        pl.BlockSpec((H, EO), lambda i: (0, 0)),
        pl.BlockSpec((1, EO), lambda i: (0, 0)),
    ]
    args = [gs, gr, ea_p, w1c, b1, w2a, w2b, b2, w3, b3]
    aliases = {}
    if prev is not None:
        in_specs.append(pl.BlockSpec((8, EO), lambda i: (0, 0)))
        args.append(prev)
        aliases = {10: 0}
    return pl.pallas_call(
        _edge_body,
        grid=(PH_TILES,),
        in_specs=in_specs,
        out_specs=pl.BlockSpec((TE, EO), lambda i: (i + off, 0)),
        out_shape=jax.ShapeDtypeStruct((EP, EO), _f32),
        input_output_aliases=aliases,
    )(*args)


ROWS_PER_TILE = NP // NS
CZ = 128


def _scatter_body(eo_hbm, dst2_hbm, out_hbm, idxs, buf0, buf1, accum,
                  sem_r0, sem_r1, sem_s0, sem_s1):
    cid = lax.axis_index("c")
    sid = lax.axis_index("s")
    wid = sid * NC + cid

    def zrow(r, carry):
        for t in range(EO // 16):
            buf0[r, pl.ds(t * 16, 16)] = jnp.zeros((16,), _f32)
        return carry

    lax.fori_loop(0, CZ, zrow, 0)
    for z in range(ROWS_PER_TILE // CZ):
        r0 = pl.multiple_of(sid * ROWS_PER_TILE + z * CZ, 8)
        pltpu.sync_copy(buf0, accum.at[pl.ds(r0, CZ)])

    pltpu.sync_copy(dst2_hbm.at[pl.ds(pl.multiple_of(wid * K2CH, 8), K2CH)], idxs)
    plsc.subcore_barrier()

    def row_off(j):
        return pl.multiple_of(wid * PER_W + j * C, 8)

    def issue_read(j, buf, sem):
        pltpu.async_copy(eo_hbm.at[pl.ds(row_off(j), C)], buf, sem)

    def wait_read(j, buf, sem):
        pltpu.make_async_copy(eo_hbm.at[pl.ds(row_off(j), C)], buf, sem).wait()

    def issue_scat(j, buf, sem):
        pltpu.async_copy(buf, accum.at[idxs.at[j]], sem, add=True)

    def wait_scat(j, buf, sem):
        pltpu.make_async_copy(buf, accum.at[idxs.at[j]], sem).wait()

    issue_read(0, buf0, sem_r0)
    issue_read(1, buf1, sem_r1)
    nb2 = K2CH // 2

    def pair(jj, carry):
        j0 = jj * 2
        j1 = j0 + 1
        wait_read(j0, buf0, sem_r0)
        issue_scat(j0, buf0, sem_s0)
        wait_read(j1, buf1, sem_r1)
        issue_scat(j1, buf1, sem_s1)
        wait_scat(j0, buf0, sem_s0)

        @pl.when(jj + 1 < nb2)
        def _():
            issue_read(j0 + 2, buf0, sem_r0)

        wait_scat(j1, buf1, sem_s1)

        @pl.when(jj + 1 < nb2)
        def _():
            issue_read(j1 + 2, buf1, sem_r1)

        return carry

    lax.fori_loop(0, nb2, pair, 0)
    plsc.subcore_barrier()

    for z in range(ROWS_PER_TILE // CZ):
        r0 = pl.multiple_of(sid * ROWS_PER_TILE + z * CZ, 8)
        pltpu.sync_copy(accum.at[pl.ds(r0, CZ)], out_hbm.at[cid].at[pl.ds(r0, CZ)])


def _scatter(e_out_p, dst2):
    mesh = plsc.VectorSubcoreMesh(core_axis_name="c", subcore_axis_name="s")
    fn = functools.partial(
        pl.kernel,
        out_type=jax.ShapeDtypeStruct((NC, NP, EO), _f32),
        mesh=mesh,
        scratch_types=[
            pltpu.VMEM((K2CH, C), jnp.int32),
            pltpu.VMEM((C, EO), _f32),
            pltpu.VMEM((C, EO), _f32),
            pltpu.VMEM_SHARED((NP, EO), _f32),
            pltpu.SemaphoreType.DMA,
            pltpu.SemaphoreType.DMA,
            pltpu.SemaphoreType.DMA,
            pltpu.SemaphoreType.DMA,
        ],
    )(_scatter_body)
    return fn(e_out_p, dst2)


def _node_body(a0_ref, a1_ref, xn1_ref, w1b_ref, b1_ref, w2_ref, b2_ref,
               w3_ref, b3_ref, out_ref):
    ie = a0_ref[...] + a1_ref[...]
    h1 = xn1_ref[...] + jnp.dot(ie, w1b_ref[...],
                                preferred_element_type=_f32) + b1_ref[...]
    h1 = jnp.maximum(h1, 0.0)
    h2 = jnp.dot(h1, w2_ref[...], preferred_element_type=_f32) + b2_ref[...]
    h2 = jnp.maximum(h2, 0.0)
    out_ref[...] = jnp.dot(h2, w3_ref[...],
                           preferred_element_type=_f32) + b3_ref[...]


def _node_mlp(acc0, acc1, xn1, wn1b, b1, w2, b2, w3, b3):
    grid = NP // TN
    return pl.pallas_call(
        _node_body,
        grid=(grid,),
        in_specs=[
            pl.BlockSpec((TN, EO), lambda i: (i, 0)),
            pl.BlockSpec((TN, EO), lambda i: (i, 0)),
            pl.BlockSpec((TN, H), lambda i: (i, 0)),
            pl.BlockSpec((EO, H), lambda i: (0, 0)),
            pl.BlockSpec((1, H), lambda i: (0, 0)),
            pl.BlockSpec((H, H), lambda i: (0, 0)),
            pl.BlockSpec((1, H), lambda i: (0, 0)),
            pl.BlockSpec((H, NO), lambda i: (0, 0)),
            pl.BlockSpec((1, NO), lambda i: (0, 0)),
        ],
        out_specs=pl.BlockSpec((TN, NO), lambda i: (i, 0)),
        out_shape=jax.ShapeDtypeStruct((NP, NO), _f32),
    )(acc0, acc1, xn1, wn1b, b1, w2, b2, w3, b3)


def kernel(x, edge_index, edge_attr, fe_W1, fe_b1, fe_W2, fe_b2, fe_W3, fe_b3,
           fn_W1, fn_b1, fn_W2, fn_b2, fn_W3, fn_b3):
    x_p = jnp.pad(x, ((0, NP - N), (0, 0)))
    dst_p = jnp.pad(edge_index[1], (0, EP - E), constant_values=N)
    ea_p = jnp.pad(edge_attr, ((0, EP - E), (0, 0)))

    spread = jnp.arange(EP - E, dtype=jnp.int32) % N
    src_g = jnp.concatenate([edge_index[0], spread])
    dst_g = jnp.concatenate([edge_index[1], spread])
    src2 = src_g.reshape(EP // C, C)
    dst2 = dst_g.reshape(EP // C, C)

    w1a = fe_W1[:DF]
    w1b = fe_W1[DF:2 * DF]
    w1c = fe_W1[2 * DF:]
    wn1a = fn_W1[:DF]
    wn1b = fn_W1[DF:]

    xs, xr, xn1 = _proj(x_p, w1a, w1b, wn1a)

    ew = (w1c, fe_b1[None, :], fe_W2[:128], fe_W2[128:], fe_b2[None, :],
          fe_W3, fe_b3[None, :])
    gs_a, gr_a = _gather(xs, xr, src2[:PH_ROWS], dst2[:PH_ROWS])
    gs_b, gr_b = _gather(xs, xr, src2[PH_ROWS:], dst2[PH_ROWS:])
    e0 = _edge_mlp(gs_a, gr_a, ea_p, *ew, phase=0)
    e_out_p = _edge_mlp(gs_b, gr_b, ea_p, *ew, phase=1, prev=e0)

    acc = _scatter(e_out_p, dst_p.reshape(EP // C, C))
    n_out_p = _node_mlp(acc[0], acc[1], xn1, wn1b, fn_b1[None, :], fn_W2,
                        fn_b2[None, :], fn_W3, fn_b3[None, :])
    return n_out_p[:N], e_out_p[:E]


# speedup vs baseline: 1.5739x; 1.0006x over previous
import functools

import jax
import jax.numpy as jnp
from jax import lax
from jax.experimental import pallas as pl
from jax.experimental.pallas import tpu as pltpu
from jax.experimental.pallas import tpu_sc as plsc

N = 10000
E = 320000
DF = 128
DE = 16
H = 256
EO = 128
NO = 128

NC = 2
NS = 16
NW = NC * NS

C = 128
K2CH = 80
PER_W = C * K2CH
EP = NW * PER_W
NP = 10240
TE = 2048
TN = 1024

NPH = 5
EPH = EP // NPH
PH_CH = K2CH // NPH
PH_ROWS = EPH // C
PH_TILES = EPH // TE

_f32 = jnp.float32
_bf16 = jnp.bfloat16


def _pack2(a):
    lo = lax.bitcast_convert_type(a[:, :128].astype(_bf16), jnp.uint16)
    hi = lax.bitcast_convert_type(a[:, 128:].astype(_bf16), jnp.uint16)
    packed = hi.astype(jnp.uint32) << 16 | lo.astype(jnp.uint32)
    return lax.bitcast_convert_type(packed, jnp.int32)


def _proj_body(x_ref, wa_ref, wb_ref, wc_ref, oa_ref, ob_ref, oc_ref):
    xb = x_ref[...]
    a = jnp.dot(xb, wa_ref[...], preferred_element_type=_f32)
    b = jnp.dot(xb, wb_ref[...], preferred_element_type=_f32)
    oa_ref[...] = _pack2(a)
    ob_ref[...] = _pack2(b)
    oc_ref[...] = jnp.dot(xb, wc_ref[...], preferred_element_type=_f32)


def _proj(x_p, w1a, w1b, wn1a):
    grid = NP // TN
    return pl.pallas_call(
        _proj_body,
        grid=(grid,),
        in_specs=[
            pl.BlockSpec((TN, DF), lambda i: (i, 0)),
            pl.BlockSpec((DF, H), lambda i: (0, 0)),
            pl.BlockSpec((DF, H), lambda i: (0, 0)),
            pl.BlockSpec((DF, H), lambda i: (0, 0)),
        ],
        out_specs=[
            pl.BlockSpec((TN, 128), lambda i: (i, 0)),
            pl.BlockSpec((TN, 128), lambda i: (i, 0)),
            pl.BlockSpec((TN, H), lambda i: (i, 0)),
        ],
        out_shape=[
            jax.ShapeDtypeStruct((NP, 128), jnp.int32),
            jax.ShapeDtypeStruct((NP, 128), jnp.int32),
            jax.ShapeDtypeStruct((NP, H), _f32),
        ],
    )(x_p, w1a, w1b, wn1a)


def _gather_body(xs_hbm, xr_hbm, src2_hbm, dst2_hbm, gs_hbm, gr_hbm,
                 idx_s, idx_d, buf_a0, buf_b0, buf_a1, buf_b1,
                 sem_a0, sem_b0, sem_a1, sem_b1, sem_o0, sem_o1):
    cid = lax.axis_index("c")
    sid = lax.axis_index("s")
    wid = sid * NC + cid
    ch0 = pl.multiple_of(wid * PH_CH, 8)

    pltpu.sync_copy(src2_hbm.at[pl.ds(ch0, PH_CH)], idx_s)
    pltpu.sync_copy(dst2_hbm.at[pl.ds(ch0, PH_CH)], idx_d)

    def row_off(j):
        return pl.multiple_of((ch0 + j) * C, 8)

    def issue(j, buf_a, buf_b, sem_a, sem_b):
        pltpu.async_copy(xs_hbm.at[idx_s.at[j]], buf_a, sem_a)
        pltpu.async_copy(xr_hbm.at[idx_d.at[j]], buf_b, sem_b)

    def wait_gather(j, buf_a, buf_b, sem_a, sem_b):
        pltpu.make_async_copy(xs_hbm.at[idx_s.at[j]], buf_a, sem_a).wait()
        pltpu.make_async_copy(xr_hbm.at[idx_d.at[j]], buf_b, sem_b).wait()

    def issue_out(j, buf_a, buf_b, sem_o):
        pltpu.async_copy(buf_a, gs_hbm.at[pl.ds(row_off(j), C)], sem_o)
        pltpu.async_copy(buf_b, gr_hbm.at[pl.ds(row_off(j), C)], sem_o)

    def wait_out(j, buf_a, buf_b, sem_o):
        pltpu.make_async_copy(buf_a, gs_hbm.at[pl.ds(row_off(j), C)], sem_o).wait()
        pltpu.make_async_copy(buf_b, gr_hbm.at[pl.ds(row_off(j), C)], sem_o).wait()

    issue(0, buf_a0, buf_b0, sem_a0, sem_b0)
    issue(1, buf_a1, buf_b1, sem_a1, sem_b1)
    nb2 = PH_CH // 2

    def pair(jj, carry):
        j0 = jj * 2
        j1 = j0 + 1
        wait_gather(j0, buf_a0, buf_b0, sem_a0, sem_b0)
        issue_out(j0, buf_a0, buf_b0, sem_o0)
        wait_gather(j1, buf_a1, buf_b1, sem_a1, sem_b1)
        issue_out(j1, buf_a1, buf_b1, sem_o1)
        wait_out(j0, buf_a0, buf_b0, sem_o0)

        @pl.when(jj + 1 < nb2)
        def _():
            issue(j0 + 2, buf_a0, buf_b0, sem_a0, sem_b0)

        wait_out(j1, buf_a1, buf_b1, sem_o1)

        @pl.when(jj + 1 < nb2)
        def _():
            issue(j1 + 2, buf_a1, buf_b1, sem_a1, sem_b1)

        return carry

    lax.fori_loop(0, nb2, pair, 0)


def _gather(xs, xr, src2, dst2):
    mesh = plsc.VectorSubcoreMesh(core_axis_name="c", subcore_axis_name="s")
    fn = functools.partial(
        pl.kernel,
        out_type=[jax.ShapeDtypeStruct((EPH, 128), jnp.int32),
                  jax.ShapeDtypeStruct((EPH, 128), jnp.int32)],
        mesh=mesh,
        scratch_types=[
            pltpu.VMEM((PH_CH, C), jnp.int32),
            pltpu.VMEM((PH_CH, C), jnp.int32),
            pltpu.VMEM((C, 128), jnp.int32),
            pltpu.VMEM((C, 128), jnp.int32),
            pltpu.VMEM((C, 128), jnp.int32),
            pltpu.VMEM((C, 128), jnp.int32),
            pltpu.SemaphoreType.DMA,
            pltpu.SemaphoreType.DMA,
            pltpu.SemaphoreType.DMA,
            pltpu.SemaphoreType.DMA,
            pltpu.SemaphoreType.DMA,
            pltpu.SemaphoreType.DMA,
        ],
    )(_gather_body)
    return fn(xs, xr, src2, dst2)


def _unpack2(gi):
    lo16 = (gi & 0xFFFF).astype(jnp.uint16)
    hi16 = ((gi >> 16) & 0xFFFF).astype(jnp.uint16)
    return (lax.bitcast_convert_type(lo16, _bf16).astype(_f32),
            lax.bitcast_convert_type(hi16, _bf16).astype(_f32))


def _edge_body(gs_ref, gr_ref, ea_ref, w1c_ref, b1_ref, w2a_ref, w2b_ref,
               b2_ref, w3_ref, b3_ref, *rest):
    out_ref = rest[-1]
    ea1 = jnp.dot(ea_ref[...], w1c_ref[...],
                  preferred_element_type=_f32) + b1_ref[...]
    sa, sb = _unpack2(gs_ref[...])
    ra, rb = _unpack2(gr_ref[...])
    h1a = jnp.maximum(sa + ra + ea1[:, :128], 0.0)
    h1b = jnp.maximum(sb + rb + ea1[:, 128:], 0.0)
    h2 = jnp.dot(h1a, w2a_ref[...], preferred_element_type=_f32) + \
         jnp.dot(h1b, w2b_ref[...], preferred_element_type=_f32) + b2_ref[...]
    h2 = jnp.maximum(h2, 0.0)
    out_ref[...] = jnp.dot(h2, w3_ref[...],
                           preferred_element_type=_f32) + b3_ref[...]


def _edge_mlp(gs, gr, ea_p, w1c, b1, w2a, w2b, b2, w3, b3, phase, prev=None):
    off = phase * PH_TILES
    in_specs = [
        pl.BlockSpec((TE, 128), lambda i: (i, 0)),
        pl.BlockSpec((TE, 128), lambda i: (i, 0)),
        pl.BlockSpec((TE, DE), lambda i: (i + off, 0)),
        pl.BlockSpec((DE, H), lambda i: (0, 0)),
        pl.BlockSpec((1, H), lambda i: (0, 0)),
        pl.BlockSpec((128, H), lambda i: (0, 0)),
        pl.BlockSpec((128, H), lambda i: (0, 0)),
        pl.BlockSpec((1, H), lambda i: (0, 0)),
        pl.BlockSpec((H, EO), lambda i: (0, 0)),
        pl.BlockSpec((1, EO), lambda i: (0, 0)),
    ]
    args = [gs, gr, ea_p, w1c, b1, w2a, w2b, b2, w3, b3]
    aliases = {}
    if prev is not None:
        in_specs.append(pl.BlockSpec((8, EO), lambda i: (0, 0)))
        args.append(prev)
        aliases = {10: 0}
    return pl.pallas_call(
        _edge_body,
        grid=(PH_TILES,),
        in_specs=in_specs,
        out_specs=pl.BlockSpec((TE, EO), lambda i: (i + off, 0)),
        out_shape=jax.ShapeDtypeStruct((EP, EO), _f32),
        input_output_aliases=aliases,
    )(*args)


ROWS_PER_TILE = NP // NS
CZ = 128


def _scatter_body(eo_hbm, dst2_hbm, out_hbm, idxs, buf0, buf1, accum,
                  sem_r0, sem_r1, sem_s0, sem_s1):
    cid = lax.axis_index("c")
    sid = lax.axis_index("s")
    wid = sid * NC + cid

    def zrow(r, carry):
        for t in range(EO // 16):
            buf0[r, pl.ds(t * 16, 16)] = jnp.zeros((16,), _f32)
        return carry

    lax.fori_loop(0, CZ, zrow, 0)
    for z in range(ROWS_PER_TILE // CZ):
        r0 = pl.multiple_of(sid * ROWS_PER_TILE + z * CZ, 8)
        pltpu.sync_copy(buf0, accum.at[pl.ds(r0, CZ)])

    pltpu.sync_copy(dst2_hbm.at[pl.ds(pl.multiple_of(wid * K2CH, 8), K2CH)], idxs)
    plsc.subcore_barrier()

    def row_off(j):
        return pl.multiple_of(wid * PER_W + j * C, 8)

    def issue_read(j, buf, sem):
        pltpu.async_copy(eo_hbm.at[pl.ds(row_off(j), C)], buf, sem)

    def wait_read(j, buf, sem):
        pltpu.make_async_copy(eo_hbm.at[pl.ds(row_off(j), C)], buf, sem).wait()

    def issue_scat(j, buf, sem):
        pltpu.async_copy(buf, accum.at[idxs.at[j]], sem, add=True)

    def wait_scat(j, buf, sem):
        pltpu.make_async_copy(buf, accum.at[idxs.at[j]], sem).wait()

    issue_read(0, buf0, sem_r0)
    issue_read(1, buf1, sem_r1)
    nb2 = K2CH // 2

    def pair(jj, carry):
        j0 = jj * 2
        j1 = j0 + 1
        wait_read(j0, buf0, sem_r0)
        issue_scat(j0, buf0, sem_s0)
        wait_read(j1, buf1, sem_r1)
        issue_scat(j1, buf1, sem_s1)
        wait_scat(j0, buf0, sem_s0)

        @pl.when(jj + 1 < nb2)
        def _():
            issue_read(j0 + 2, buf0, sem_r0)

        wait_scat(j1, buf1, sem_s1)

        @pl.when(jj + 1 < nb2)
        def _():
            issue_read(j1 + 2, buf1, sem_r1)

        return carry

    lax.fori_loop(0, nb2, pair, 0)
    plsc.subcore_barrier()

    for z in range(ROWS_PER_TILE // CZ):
        r0 = pl.multiple_of(sid * ROWS_PER_TILE + z * CZ, 8)
        pltpu.sync_copy(accum.at[pl.ds(r0, CZ)], out_hbm.at[cid].at[pl.ds(r0, CZ)])


def _scatter(e_out_p, dst2):
    mesh = plsc.VectorSubcoreMesh(core_axis_name="c", subcore_axis_name="s")
    fn = functools.partial(
        pl.kernel,
        out_type=jax.ShapeDtypeStruct((NC, NP, EO), _f32),
        mesh=mesh,
        scratch_types=[
            pltpu.VMEM((K2CH, C), jnp.int32),
            pltpu.VMEM((C, EO), _f32),
            pltpu.VMEM((C, EO), _f32),
            pltpu.VMEM_SHARED((NP, EO), _f32),
            pltpu.SemaphoreType.DMA,
            pltpu.SemaphoreType.DMA,
            pltpu.SemaphoreType.DMA,
            pltpu.SemaphoreType.DMA,
        ],
    )(_scatter_body)
    return fn(e_out_p, dst2)


def _node_body(a0_ref, a1_ref, xn1_ref, w1b_ref, b1_ref, w2_ref, b2_ref,
               w3_ref, b3_ref, out_ref):
    ie = a0_ref[...] + a1_ref[...]
    h1 = xn1_ref[...] + jnp.dot(ie, w1b_ref[...],
                                preferred_element_type=_f32) + b1_ref[...]
    h1 = jnp.maximum(h1, 0.0)
    h2 = jnp.dot(h1, w2_ref[...], preferred_element_type=_f32) + b2_ref[...]
    h2 = jnp.maximum(h2, 0.0)
    out_ref[...] = jnp.dot(h2, w3_ref[...],
                           preferred_element_type=_f32) + b3_ref[...]


def _node_mlp(acc0, acc1, xn1, wn1b, b1, w2, b2, w3, b3):
    grid = NP // TN
    return pl.pallas_call(
        _node_body,
        grid=(grid,),
        in_specs=[
            pl.BlockSpec((TN, EO), lambda i: (i, 0)),
            pl.BlockSpec((TN, EO), lambda i: (i, 0)),
            pl.BlockSpec((TN, H), lambda i: (i, 0)),
            pl.BlockSpec((EO, H), lambda i: (0, 0)),
            pl.BlockSpec((1, H), lambda i: (0, 0)),
            pl.BlockSpec((H, H), lambda i: (0, 0)),
            pl.BlockSpec((1, H), lambda i: (0, 0)),
            pl.BlockSpec((H, NO), lambda i: (0, 0)),
            pl.BlockSpec((1, NO), lambda i: (0, 0)),
        ],
        out_specs=pl.BlockSpec((TN, NO), lambda i: (i, 0)),
        out_shape=jax.ShapeDtypeStruct((NP, NO), _f32),
    )(acc0, acc1, xn1, wn1b, b1, w2, b2, w3, b3)


def kernel(x, edge_index, edge_attr, fe_W1, fe_b1, fe_W2, fe_b2, fe_W3, fe_b3,
           fn_W1, fn_b1, fn_W2, fn_b2, fn_W3, fn_b3):
    x_p = jnp.pad(x, ((0, NP - N), (0, 0)))
    dst_p = jnp.pad(edge_index[1], (0, EP - E), constant_values=N)
    ea_p = jnp.pad(edge_attr, ((0, EP - E), (0, 0)))

    spread = jnp.arange(EP - E, dtype=jnp.int32) % N
    src_g = jnp.concatenate([edge_index[0], spread])
    dst_g = jnp.concatenate([edge_index[1], spread])
    src2 = src_g.reshape(EP // C, C)
    dst2 = dst_g.reshape(EP // C, C)

    w1a = fe_W1[:DF]
    w1b = fe_W1[DF:2 * DF]
    w1c = fe_W1[2 * DF:]
    wn1a = fn_W1[:DF]
    wn1b = fn_W1[DF:]

    xs, xr, xn1 = _proj(x_p, w1a, w1b, wn1a)

    ew = (w1c, fe_b1[None, :], fe_W2[:128], fe_W2[128:], fe_b2[None, :],
          fe_W3, fe_b3[None, :])
    e_out_p = None
    for p in range(NPH):
        rows = slice(p * PH_ROWS, (p + 1) * PH_ROWS)
        gs_p, gr_p = _gather(xs, xr, src2[rows], dst2[rows])
        e_out_p = _edge_mlp(gs_p, gr_p, ea_p, *ew, phase=p, prev=e_out_p)

    acc = _scatter(e_out_p, dst_p.reshape(EP // C, C))
    n_out_p = _node_mlp(acc[0], acc[1], xn1, wn1b, fn_b1[None, :], fn_W2,
                        fn_b2[None, :], fn_W3, fn_b3[None, :])
    return n_out_p[:N], e_out_p[:E]
